# Initial kernel scaffold; baseline (speedup 1.0000x reference)
import jax
import jax.numpy as jnp
from jax.experimental import pallas as pl


def kernel(positions, hash_tables):
    raise NotImplementedError("write your pallas kernel here")


# speedup vs baseline: 3.4816x; 3.4816x over previous
import functools
import math

import jax
import jax.numpy as jnp
from jax import lax
from jax.experimental import pallas as pl
from jax.experimental.pallas import tpu as pltpu
from jax.experimental.pallas import tpu_sc as plsc

_NUM_LEVELS = 8
_TABLE = 1048576
_F = 4
_BASE_RES = 16
_FINEST_RES = 512
_N = 262144

_b = math.exp(math.log(_FINEST_RES / _BASE_RES) / (_NUM_LEVELS - 1))
_RES = [int(_BASE_RES * _b ** l) for l in range(_NUM_LEVELS)]
_P2 = -1640531535
_P3 = 805459861
_MASK = _TABLE - 1

_NC, _NS = 2, 16
_NW = _NC * _NS
_PPT = _N // _NW
_PB = 128
_NBLK = _PPT // _PB
_NG = _PB // 16
_LC = _NUM_LEVELS * 8
_TW = _NUM_LEVELS * _TABLE * _F
_EHI = _N // _PB


def _i32(v):
    return jnp.int32(v)


def _f32(v):
    return jnp.float32(v)


def _body(xs_hbm, ys_hbm, zs_hbm, tbl_hbm, out_hbm,
          xs_v, ys_v, zs_v, idx_v, t_v, rows_v, ob_v, sem):
    wid = (lax.convert_element_type(lax.axis_index("s"), jnp.int32) * _i32(_NC)
           + lax.convert_element_type(lax.axis_index("c"), jnp.int32))
    base = wid * _i32(_PPT)
    pltpu.sync_copy(xs_hbm.at[pl.ds(base, _PPT)], xs_v)
    pltpu.sync_copy(ys_hbm.at[pl.ds(base, _PPT)], ys_v)
    pltpu.sync_copy(zs_hbm.at[pl.ds(base, _PPT)], zs_v)

    @pl.loop(jnp.int32(0), jnp.int32(_NBLK), step=jnp.int32(1))
    def _blk(blk):
        boff = blk * _i32(_PB)

        @pl.loop(jnp.int32(0), jnp.int32(_NG), step=jnp.int32(1))
        def _g1(g):
            off = boff + g * _i32(16)
            loc = g * _i32(16)
            x = jnp.minimum(jnp.maximum(xs_v[pl.ds(off, 16)], _f32(0.0)), _f32(1.0))
            y = jnp.minimum(jnp.maximum(ys_v[pl.ds(off, 16)], _f32(0.0)), _f32(1.0))
            z = jnp.minimum(jnp.maximum(zs_v[pl.ds(off, 16)], _f32(0.0)), _f32(1.0))
            for l in range(_NUM_LEVELS):
                r = _f32(_RES[l])
                sx = x * r
                sy = y * r
                sz = z * r
                fx = sx.astype(jnp.int32)
                fy = sy.astype(jnp.int32)
                fz = sz.astype(jnp.int32)
                t_v[3 * l + 0, pl.ds(loc, 16)] = sx - fx.astype(jnp.float32)
                t_v[3 * l + 1, pl.ds(loc, 16)] = sy - fy.astype(jnp.float32)
                t_v[3 * l + 2, pl.ds(loc, 16)] = sz - fz.astype(jnp.float32)
                cx = jnp.minimum(fx, _i32(_RES[l] - 1))
                cy = jnp.minimum(fy, _i32(_RES[l] - 1))
                cz = jnp.minimum(fz, _i32(_RES[l] - 1))
                a = [cx, cx + _i32(1)]
                b0 = cy * _i32(_P2)
                b = [b0, b0 + _i32(_P2)]
                c0 = cz * _i32(_P3)
                c = [c0, c0 + _i32(_P3)]
                loff = _i32(l * _TABLE * _F)
                for i in range(2):
                    for j in range(2):
                        s = a[i] + b[j]
                        for k in range(2):
                            hm = (s + c[k]) & _i32(_MASK)
                            w = ((hm << _i32(2)) - (hm & _i32(127)) * _i32(3)
                                 + loff)
                            idx_v[8 * l + 4 * i + 2 * j + k, pl.ds(loc, 16)] = w

        @pl.loop(jnp.int32(0), jnp.int32(_LC), step=jnp.int32(1))
        def _fire(i):
            for f in range(_F):
                pltpu.async_copy(
                    tbl_hbm.at[pl.ds(f * 128, _TW - f * 128)].at[idx_v.at[i]],
                    rows_v.at[i * _i32(_F) + _i32(f)], sem)

        @pl.loop(jnp.int32(0), jnp.int32(_LC * _F), step=jnp.int32(1))
        def _drain(i):
            pltpu.make_async_copy(tbl_hbm.at[pl.ds(0, _PB)],
                                  rows_v.at[i], sem).wait()

        @pl.loop(jnp.int32(0), jnp.int32(_NG), step=jnp.int32(1))
        def _g2(g):
            loc = g * _i32(16)
            for l in range(_NUM_LEVELS):
                tx = t_v[3 * l + 0, pl.ds(loc, 16)]
                ty = t_v[3 * l + 1, pl.ds(loc, 16)]
                tz = t_v[3 * l + 2, pl.ds(loc, 16)]
                wx = [_f32(1.0) - tx, tx]
                wy = [_f32(1.0) - ty, ty]
                wz = [_f32(1.0) - tz, tz]
                wc = []
                for i in range(2):
                    for j in range(2):
                        wij = wy[i] * wz[j]
                        for k in range(2):
                            wc.append(wij * wx[k])
                for f in range(_F):
                    acc = None
                    for cidx in range(8):
                        feat = rows_v[(8 * l + cidx) * _F + f, pl.ds(loc, 16)]
                        term = feat * wc[cidx]
                        acc = term if acc is None else acc + term
                    ob_v[4 * l + f, pl.ds(loc, 16)] = acc

        et = wid * _i32(_NBLK) + blk
        for chi in range(4):
            pltpu.sync_copy(ob_v.at[pl.ds(chi * 8, 8)],
                            out_hbm.at[_i32(chi), et])


_encode = functools.partial(
    pl.kernel,
    out_type=jax.ShapeDtypeStruct((4, _EHI, 8, _PB), jnp.float32),
    mesh=plsc.VectorSubcoreMesh(core_axis_name="c", subcore_axis_name="s",
                                num_cores=_NC, num_subcores=_NS),
    compiler_params=pltpu.CompilerParams(needs_layout_passes=False,
                                         use_tc_tiling_on_sc=False),
    scratch_types=[
        pltpu.VMEM((_PPT,), jnp.float32),
        pltpu.VMEM((_PPT,), jnp.float32),
        pltpu.VMEM((_PPT,), jnp.float32),
        pltpu.VMEM((_LC, _PB), jnp.int32),
        pltpu.VMEM((3 * _NUM_LEVELS, _PB), jnp.float32),
        pltpu.VMEM((_LC * _F, _PB), jnp.float32),
        pltpu.VMEM((_NUM_LEVELS * _F, _PB), jnp.float32),
        pltpu.SemaphoreType.DMA,
    ],
)(_body)


@jax.jit
def _impl(positions, hash_tables):
    xs = positions[:, 0]
    ys = positions[:, 1]
    zs = positions[:, 2]
    tblv = (hash_tables.reshape(_NUM_LEVELS, _TABLE // 128, 128, _F)
            .transpose(0, 1, 3, 2).reshape(-1))
    o = _encode(xs, ys, zs, tblv)
    return o.transpose(1, 3, 0, 2).reshape(_N, _NUM_LEVELS * _F)


def kernel(positions, hash_tables):
    return _impl(positions, hash_tables)


# speedup vs baseline: 3.5692x; 1.0252x over previous
import functools
import math

import jax
import jax.numpy as jnp
from jax import lax
from jax.experimental import pallas as pl
from jax.experimental.pallas import tpu as pltpu
from jax.experimental.pallas import tpu_sc as plsc

_NUM_LEVELS = 8
_TABLE = 1048576
_F = 4
_BASE_RES = 16
_FINEST_RES = 512
_N = 262144

_b = math.exp(math.log(_FINEST_RES / _BASE_RES) / (_NUM_LEVELS - 1))
_RES = [int(_BASE_RES * _b ** l) for l in range(_NUM_LEVELS)]
_P2 = -1640531535
_P3 = 805459861
_MASK = _TABLE - 1

_NC, _NS = 2, 16
_NW = _NC * _NS
_PPT = _N // _NW
_PB = 128
_NBLK = _PPT // _PB
_NG = _PB // 16
_LC = _NUM_LEVELS * 8
_RW = _LC * _F * _PB
_TW = _NUM_LEVELS * _TABLE * _F
_EHI = _N // _PB


def _i32(v):
    return jnp.int32(v)


def _f32(v):
    return jnp.float32(v)


def _body(xs_hbm, ys_hbm, zs_hbm, tbl_hbm, out_hbm,
          pos_v, idx0, idx1, t0, t1, rows0, rows1, ob_v,
          sem0, sem1):
    wid = (lax.convert_element_type(lax.axis_index("s"), jnp.int32) * _i32(_NC)
           + lax.convert_element_type(lax.axis_index("c"), jnp.int32))
    base = wid * _i32(_PPT)

    bufs = ((idx0, t0, rows0, sem0), (idx1, t1, rows1, sem1))

    def pass1(boff, par):
        idx_v, t_v, _, _ = bufs[par]
        gbase = base + boff
        pltpu.sync_copy(xs_hbm.at[pl.ds(gbase, _PB)], pos_v.at[_i32(0)])
        pltpu.sync_copy(ys_hbm.at[pl.ds(gbase, _PB)], pos_v.at[_i32(1)])
        pltpu.sync_copy(zs_hbm.at[pl.ds(gbase, _PB)], pos_v.at[_i32(2)])

        @pl.loop(jnp.int32(0), jnp.int32(_NG), step=jnp.int32(1))
        def _g1(g):
            loc = g * _i32(16)
            x = jnp.minimum(jnp.maximum(pos_v[0, pl.ds(loc, 16)], _f32(0.0)),
                            _f32(1.0))
            y = jnp.minimum(jnp.maximum(pos_v[1, pl.ds(loc, 16)], _f32(0.0)),
                            _f32(1.0))
            z = jnp.minimum(jnp.maximum(pos_v[2, pl.ds(loc, 16)], _f32(0.0)),
                            _f32(1.0))
            for l in range(_NUM_LEVELS):
                r = _f32(_RES[l])
                sx = x * r
                sy = y * r
                sz = z * r
                fx = sx.astype(jnp.int32)
                fy = sy.astype(jnp.int32)
                fz = sz.astype(jnp.int32)
                t_v[3 * l + 0, pl.ds(loc, 16)] = sx - fx.astype(jnp.float32)
                t_v[3 * l + 1, pl.ds(loc, 16)] = sy - fy.astype(jnp.float32)
                t_v[3 * l + 2, pl.ds(loc, 16)] = sz - fz.astype(jnp.float32)
                cx = jnp.minimum(fx, _i32(_RES[l] - 1))
                cy = jnp.minimum(fy, _i32(_RES[l] - 1))
                cz = jnp.minimum(fz, _i32(_RES[l] - 1))
                a = [cx, cx + _i32(1)]
                b0 = cy * _i32(_P2)
                b = [b0, b0 + _i32(_P2)]
                c0 = cz * _i32(_P3)
                c = [c0, c0 + _i32(_P3)]
                loff = _i32(l * _TABLE * _F)
                for i in range(2):
                    for j in range(2):
                        s = a[i] + b[j]
                        for k in range(2):
                            hm = (s + c[k]) & _i32(_MASK)
                            w = ((hm << _i32(2)) - (hm & _i32(127)) * _i32(3)
                                 + loff)
                            idx_v[8 * l + 4 * i + 2 * j + k, pl.ds(loc, 16)] = w

    def fire(par):
        idx_v, _, rows_v, sem = bufs[par]

        @pl.loop(jnp.int32(0), jnp.int32(_LC), step=jnp.int32(1))
        def _fire(i):
            for f in range(_F):
                pltpu.async_copy(
                    tbl_hbm.at[pl.ds(f * 128, _TW - f * 128)].at[idx_v.at[i]],
                    rows_v.at[pl.ds((i * _i32(_F) + _i32(f)) * _i32(_PB), _PB)],
                    sem)

    def drain(par):
        _, _, rows_v, sem = bufs[par]
        pltpu.make_async_copy(tbl_hbm.at[pl.ds(0, _RW)], rows_v, sem).wait()

    def blend_out(boff, blk, par):
        _, t_v, rows_v, _ = bufs[par]

        @pl.loop(jnp.int32(0), jnp.int32(_NG), step=jnp.int32(1))
        def _g2(g):
            loc = g * _i32(16)
            for l in range(_NUM_LEVELS):
                tx = t_v[3 * l + 0, pl.ds(loc, 16)]
                ty = t_v[3 * l + 1, pl.ds(loc, 16)]
                tz = t_v[3 * l + 2, pl.ds(loc, 16)]
                wx = [_f32(1.0) - tx, tx]
                wy = [_f32(1.0) - ty, ty]
                wz = [_f32(1.0) - tz, tz]
                wc = []
                for i in range(2):
                    for j in range(2):
                        wij = wy[i] * wz[j]
                        for k in range(2):
                            wc.append(wij * wx[k])
                for f in range(_F):
                    acc = None
                    for cidx in range(8):
                        foff = ((8 * l + cidx) * _F + f) * _PB
                        feat = rows_v[pl.ds(_i32(foff) + loc, 16)]
                        term = feat * wc[cidx]
                        acc = term if acc is None else acc + term
                    ob_v[4 * l + f, pl.ds(loc, 16)] = acc

        et = wid * _i32(_NBLK) + blk
        for chi in range(4):
            pltpu.sync_copy(ob_v.at[pl.ds(chi * 8, 8)],
                            out_hbm.at[_i32(chi), et])

    pass1(_i32(0), 0)
    fire(0)

    @pl.loop(jnp.int32(0), jnp.int32(_NBLK // 2), step=jnp.int32(1))
    def _bb(bb):
        b0 = bb * _i32(2)
        b1 = b0 + _i32(1)
        pass1(b1 * _i32(_PB), 1)
        fire(1)
        drain(0)
        blend_out(b0 * _i32(_PB), b0, 0)

        @pl.when(bb < _i32(_NBLK // 2 - 1))
        def _():
            pass1((b0 + _i32(2)) * _i32(_PB), 0)
            fire(0)

        drain(1)
        blend_out(b1 * _i32(_PB), b1, 1)


_encode = functools.partial(
    pl.kernel,
    out_type=jax.ShapeDtypeStruct((4, _EHI, 8, _PB), jnp.float32),
    mesh=plsc.VectorSubcoreMesh(core_axis_name="c", subcore_axis_name="s",
                                num_cores=_NC, num_subcores=_NS),
    compiler_params=pltpu.CompilerParams(needs_layout_passes=False,
                                         use_tc_tiling_on_sc=False),
    scratch_types=[
        pltpu.VMEM((3, _PB), jnp.float32),
        pltpu.VMEM((_LC, _PB), jnp.int32),
        pltpu.VMEM((_LC, _PB), jnp.int32),
        pltpu.VMEM((3 * _NUM_LEVELS, _PB), jnp.float32),
        pltpu.VMEM((3 * _NUM_LEVELS, _PB), jnp.float32),
        pltpu.VMEM((_RW,), jnp.float32),
        pltpu.VMEM((_RW,), jnp.float32),
        pltpu.VMEM((_NUM_LEVELS * _F, _PB), jnp.float32),
        pltpu.SemaphoreType.DMA,
        pltpu.SemaphoreType.DMA,
    ],
)(_body)


@jax.jit
def _impl(positions, hash_tables):
    xs = positions[:, 0]
    ys = positions[:, 1]
    zs = positions[:, 2]
    tblv = (hash_tables.reshape(_NUM_LEVELS, _TABLE // 128, 128, _F)
            .transpose(0, 1, 3, 2).reshape(-1))
    o = _encode(xs, ys, zs, tblv)
    return o.transpose(1, 3, 0, 2).reshape(_N, _NUM_LEVELS * _F)


def kernel(positions, hash_tables):
    return _impl(positions, hash_tables)


# speedup vs baseline: 4.0601x; 1.1375x over previous
import functools
import math

import jax
import jax.numpy as jnp
from jax import lax
from jax.experimental import pallas as pl
from jax.experimental.pallas import tpu as pltpu
from jax.experimental.pallas import tpu_sc as plsc

_NUM_LEVELS = 8
_TABLE = 1048576
_F = 4
_BASE_RES = 16
_FINEST_RES = 512
_N = 262144

_b = math.exp(math.log(_FINEST_RES / _BASE_RES) / (_NUM_LEVELS - 1))
_RES = [int(_BASE_RES * _b ** l) for l in range(_NUM_LEVELS)]
_P2 = -1640531535
_P3 = 805459861
_MASK = _TABLE - 1

_NC, _NS = 2, 16
_NW = _NC * _NS
_PPT = _N // _NW
_PB = 128
_NBLK = _PPT // _PB
_NG = _PB // 16
_LC = (_NUM_LEVELS - 1) * 8
_L0CELLS = 17 * 17 * 17
_L0PAD = 4992
_L0W = _L0PAD * _F
_RW = _LC * _F * _PB
_TW = _NUM_LEVELS * _TABLE * _F
_EHI = _N // _PB

_MESH = plsc.VectorSubcoreMesh(core_axis_name="c", subcore_axis_name="s",
                               num_cores=_NC, num_subcores=_NS)
_CP = pltpu.CompilerParams(needs_layout_passes=False, use_tc_tiling_on_sc=False)


def _i32(v):
    return jnp.int32(v)


def _f32(v):
    return jnp.float32(v)


def _body(xs_hbm, ys_hbm, zs_hbm, tbl_hbm, out_hbm,
          pos_v, l0_v, cell0, cell1, idx0, idx1, t0, t1, rows0, rows1, ob_v,
          sem0, sem1):
    wid = (lax.convert_element_type(lax.axis_index("s"), jnp.int32) * _i32(_NC)
           + lax.convert_element_type(lax.axis_index("c"), jnp.int32))
    base = wid * _i32(_PPT)
    iota = lax.iota(jnp.int32, 16)

    bufs = ((idx0, t0, rows0, sem0, cell0), (idx1, t1, rows1, sem1, cell1))

    @pl.loop(jnp.int32(0), jnp.int32(_L0PAD // 16), step=jnp.int32(1))
    def _l0idx(v):
        c = jnp.minimum(v * _i32(16) + iota, _i32(_L0CELLS - 1))
        cx = c // _i32(289)
        r0 = c - cx * _i32(289)
        cy = r0 // _i32(17)
        cz = r0 - cy * _i32(17)
        hm = (cx + cy * _i32(_P2) + cz * _i32(_P3)) & _i32(_MASK)
        w = (hm << _i32(2)) - (hm & _i32(127)) * _i32(3)
        j = lax.div(v, _i32(8))
        loc = (v - j * _i32(8)) * _i32(16)
        idx0[j, pl.ds(loc, 16)] = w

    @pl.loop(jnp.int32(0), jnp.int32(_L0PAD // 128), step=jnp.int32(1))
    def _l0fire(j):
        for f in range(_F):
            pltpu.async_copy(
                tbl_hbm.at[pl.ds(f * 128, _TW - f * 128)].at[idx0.at[j]],
                l0_v.at[pl.ds(j * _i32(512) + _i32(f * 128), _PB)], sem0)

    pltpu.make_async_copy(tbl_hbm.at[pl.ds(0, _L0W)], l0_v, sem0).wait()

    def pass1(boff, par):
        idx_v, t_v, _, _, cell_v = bufs[par]
        gbase = base + boff
        pltpu.sync_copy(xs_hbm.at[pl.ds(gbase, _PB)], pos_v.at[_i32(0)])
        pltpu.sync_copy(ys_hbm.at[pl.ds(gbase, _PB)], pos_v.at[_i32(1)])
        pltpu.sync_copy(zs_hbm.at[pl.ds(gbase, _PB)], pos_v.at[_i32(2)])

        @pl.loop(jnp.int32(0), jnp.int32(_NG), step=jnp.int32(1))
        def _g1(g):
            loc = g * _i32(16)
            x = jnp.minimum(jnp.maximum(pos_v[0, pl.ds(loc, 16)], _f32(0.0)),
                            _f32(1.0))
            y = jnp.minimum(jnp.maximum(pos_v[1, pl.ds(loc, 16)], _f32(0.0)),
                            _f32(1.0))
            z = jnp.minimum(jnp.maximum(pos_v[2, pl.ds(loc, 16)], _f32(0.0)),
                            _f32(1.0))
            for l in range(_NUM_LEVELS):
                r = _f32(_RES[l])
                sx = x * r
                sy = y * r
                sz = z * r
                fx = sx.astype(jnp.int32)
                fy = sy.astype(jnp.int32)
                fz = sz.astype(jnp.int32)
                t_v[3 * l + 0, pl.ds(loc, 16)] = sx - fx.astype(jnp.float32)
                t_v[3 * l + 1, pl.ds(loc, 16)] = sy - fy.astype(jnp.float32)
                t_v[3 * l + 2, pl.ds(loc, 16)] = sz - fz.astype(jnp.float32)
                cx = jnp.minimum(fx, _i32(_RES[l] - 1))
                cy = jnp.minimum(fy, _i32(_RES[l] - 1))
                cz = jnp.minimum(fz, _i32(_RES[l] - 1))
                a = [cx, cx + _i32(1)]
                b0 = cy * _i32(_P2)
                b = [b0, b0 + _i32(_P2)]
                c0 = cz * _i32(_P3)
                c = [c0, c0 + _i32(_P3)]
                if l == 0:
                    cell_v[0, pl.ds(loc, 16)] = (cx * _i32(17) + cy) * _i32(17) + cz
                    continue
                loff = _i32(l * _TABLE * _F)
                for i in range(2):
                    for j in range(2):
                        s = a[i] + b[j]
                        for k in range(2):
                            hm = (s + c[k]) & _i32(_MASK)
                            w = ((hm << _i32(2)) - (hm & _i32(127)) * _i32(3)
                                 + loff)
                            idx_v[8 * (l - 1) + 4 * i + 2 * j + k,
                                  pl.ds(loc, 16)] = w

    def fire(par):
        idx_v, _, rows_v, sem, _ = bufs[par]

        @pl.loop(jnp.int32(0), jnp.int32(_LC), step=jnp.int32(1))
        def _fire(i):
            for f in range(_F):
                pltpu.async_copy(
                    tbl_hbm.at[pl.ds(f * 128, _TW - f * 128)].at[idx_v.at[i]],
                    rows_v.at[pl.ds((i * _i32(_F) + _i32(f)) * _i32(_PB), _PB)],
                    sem)

    def drain(par):
        _, _, rows_v, sem, _ = bufs[par]
        pltpu.make_async_copy(tbl_hbm.at[pl.ds(0, _RW)], rows_v, sem).wait()

    def blend_out(boff, blk, par):
        _, t_v, rows_v, _, cell_v = bufs[par]

        @pl.loop(jnp.int32(0), jnp.int32(_NG), step=jnp.int32(1))
        def _g2(g):
            loc = g * _i32(16)
            for l in range(_NUM_LEVELS):
                tx = t_v[3 * l + 0, pl.ds(loc, 16)]
                ty = t_v[3 * l + 1, pl.ds(loc, 16)]
                tz = t_v[3 * l + 2, pl.ds(loc, 16)]
                wx = [_f32(1.0) - tx, tx]
                wy = [_f32(1.0) - ty, ty]
                wz = [_f32(1.0) - tz, tz]
                wc = []
                for i in range(2):
                    for j in range(2):
                        wij = wy[i] * wz[j]
                        for k in range(2):
                            wc.append(wij * wx[k])
                if l == 0:
                    cb = cell_v[0, pl.ds(loc, 16)]
                    accs = [None] * _F
                    for cidx in range(8):
                        i, j, k = cidx >> 2, (cidx >> 1) & 1, cidx & 1
                        cc = cb + _i32(i * 289 + j * 17 + k)
                        w0 = (cc << _i32(2)) - (cc & _i32(127)) * _i32(3)
                        for f in range(_F):
                            feat = plsc.load_gather(l0_v, [w0 + _i32(f * 128)])
                            term = feat * wc[cidx]
                            accs[f] = term if accs[f] is None else accs[f] + term
                    for f in range(_F):
                        ob_v[f, pl.ds(loc, 16)] = accs[f]
                    continue
                for f in range(_F):
                    acc = None
                    for cidx in range(8):
                        foff = ((8 * (l - 1) + cidx) * _F + f) * _PB
                        feat = rows_v[pl.ds(_i32(foff) + loc, 16)]
                        term = feat * wc[cidx]
                        acc = term if acc is None else acc + term
                    ob_v[4 * l + f, pl.ds(loc, 16)] = acc

        et = wid * _i32(_NBLK) + blk
        for chi in range(4):
            pltpu.sync_copy(ob_v.at[pl.ds(chi * 8, 8)],
                            out_hbm.at[_i32(chi), et])

    pass1(_i32(0), 0)
    fire(0)

    @pl.loop(jnp.int32(0), jnp.int32(_NBLK // 2), step=jnp.int32(1))
    def _bb(bb):
        b0 = bb * _i32(2)
        b1 = b0 + _i32(1)
        pass1(b1 * _i32(_PB), 1)
        fire(1)
        drain(0)
        blend_out(b0 * _i32(_PB), b0, 0)

        @pl.when(bb < _i32(_NBLK // 2 - 1))
        def _():
            pass1((b0 + _i32(2)) * _i32(_PB), 0)
            fire(0)

        drain(1)
        blend_out(b1 * _i32(_PB), b1, 1)


_encode = functools.partial(
    pl.kernel,
    out_type=jax.ShapeDtypeStruct((4, _EHI, 8, _PB), jnp.float32),
    mesh=_MESH,
    compiler_params=_CP,
    scratch_types=[
        pltpu.VMEM((3, _PB), jnp.float32),
        pltpu.VMEM((_L0W,), jnp.float32),
        pltpu.VMEM((1, _PB), jnp.int32),
        pltpu.VMEM((1, _PB), jnp.int32),
        pltpu.VMEM((_LC, _PB), jnp.int32),
        pltpu.VMEM((_LC, _PB), jnp.int32),
        pltpu.VMEM((3 * _NUM_LEVELS, _PB), jnp.float32),
        pltpu.VMEM((3 * _NUM_LEVELS, _PB), jnp.float32),
        pltpu.VMEM((_RW,), jnp.float32),
        pltpu.VMEM((_RW,), jnp.float32),
        pltpu.VMEM((_NUM_LEVELS * _F, _PB), jnp.float32),
        pltpu.SemaphoreType.DMA,
        pltpu.SemaphoreType.DMA,
    ],
)(_body)


@jax.jit
def _impl(positions, hash_tables):
    xs = positions[:, 0]
    ys = positions[:, 1]
    zs = positions[:, 2]
    tblv = (hash_tables.reshape(_NUM_LEVELS, _TABLE // 128, 128, _F)
            .transpose(0, 1, 3, 2).reshape(-1))
    o = _encode(xs, ys, zs, tblv)
    return o.transpose(1, 3, 0, 2).reshape(_N, _NUM_LEVELS * _F)


def kernel(positions, hash_tables):
    return _impl(positions, hash_tables)
